# baseline (speedup 1.0000x reference)
import functools

import jax
import jax.numpy as jnp
from jax import lax
from jax.experimental import pallas as pl
from jax.experimental.pallas import tpu as pltpu
from jax.experimental.pallas import tpu_sc as plsc

_MAXZ = 84
_NP = 24
_N = 1048576
_NC, _NS, _L = 2, 16, 16
_NW = _NC * _NS
_APW = _N // _NW
_CHUNK = 2048
_NCHUNK = _APW // _CHUNK
_TBL = _NP * _MAXZ


def _build_sc_gather():
    mesh = plsc.VectorSubcoreMesh(
        core_axis_name="c", subcore_axis_name="s",
        num_cores=_NC, num_subcores=_NS)

    @functools.partial(
        pl.kernel,
        out_type=jax.ShapeDtypeStruct((_NP, _N), jnp.float32),
        mesh=mesh,
        compiler_params=pltpu.CompilerParams(
            needs_layout_passes=False, use_tc_tiling_on_sc=True),
        scratch_types=[
            pltpu.VMEM((_NP, _MAXZ), jnp.float32),
            pltpu.VMEM((_CHUNK,), jnp.int32),
            pltpu.VMEM((_CHUNK,), jnp.int32),
            pltpu.VMEM((_NP, _CHUNK), jnp.float32),
            pltpu.VMEM((_NP, _CHUNK), jnp.float32),
            pltpu.SemaphoreType.DMA,
            pltpu.SemaphoreType.DMA,
            pltpu.SemaphoreType.DMA,
            pltpu.SemaphoreType.DMA,
        ],
    )
    def sc_gather(z_hbm, pt_hbm, out_hbm, pt_v, z_a, z_b, blk_a, blk_b,
                  osem_a, osem_b, zsem_a, zsem_b):
        wid = lax.axis_index("s") * _NC + lax.axis_index("c")
        atom0 = wid * _APW
        pltpu.sync_copy(pt_hbm, pt_v)
        pltpu.async_copy(z_hbm.at[pl.ds(atom0, _CHUNK)], z_a, zsem_a)

        def compute(blk, z_v):
            @plsc.parallel_loop(0, _CHUNK // _L, unroll=1)
            def group_body(a):
                aoff = a * _L
                zvec = z_v[pl.ds(aoff, _L)]
                for j in range(_NP):
                    jvec = jnp.full((_L,), j, dtype=jnp.int32)
                    vals = plsc.load_gather(pt_v, [jvec, zvec])
                    blk[j, pl.ds(aoff, _L)] = vals

        def pipe_body(g2, carry):
            for blk, osem, z_cur, zsem_cur, z_nxt, zsem_nxt, cidx in (
                    (blk_a, osem_a, z_a, zsem_a, z_b, zsem_b, 2 * g2),
                    (blk_b, osem_b, z_b, zsem_b, z_a, zsem_a, 2 * g2 + 1)):
                @pl.when(cidx + 1 < _NCHUNK)
                def _prefetch():
                    pltpu.async_copy(
                        z_hbm.at[pl.ds(atom0 + (cidx + 1) * _CHUNK, _CHUNK)],
                        z_nxt, zsem_nxt)

                pltpu.make_async_copy(
                    z_hbm.at[pl.ds(atom0, _CHUNK)], z_cur, zsem_cur).wait()

                @pl.when(g2 > 0)
                def _wait():
                    pltpu.make_async_copy(
                        blk, out_hbm.at[:, pl.ds(atom0, _CHUNK)], osem).wait()

                compute(blk, z_cur)
                pltpu.async_copy(
                    blk, out_hbm.at[:, pl.ds(atom0 + cidx * _CHUNK, _CHUNK)],
                    osem)
            return carry

        lax.fori_loop(0, _NCHUNK // 2, pipe_body, 0)
        pltpu.make_async_copy(
            blk_a, out_hbm.at[:, pl.ds(atom0, _CHUNK)], osem_a).wait()
        pltpu.make_async_copy(
            blk_b, out_hbm.at[:, pl.ds(atom0, _CHUNK)], osem_b).wait()

    return sc_gather


_SC_GATHER = _build_sc_gather()


def kernel(Z, p, alpha, chi):
    z32 = Z.astype(jnp.int32)
    outT = _SC_GATHER(z32, p.T)
    return (outT.T, alpha, chi)


# speedup vs baseline: 1.1072x; 1.1072x over previous
import functools

import jax
import jax.numpy as jnp
from jax import lax
from jax.experimental import pallas as pl
from jax.experimental.pallas import tpu as pltpu
from jax.experimental.pallas import tpu_sc as plsc

_MAXZ = 84
_NP = 24
_N = 1048576
_NC, _NS, _L = 2, 16, 16
_NW = _NC * _NS
_APW = _N // _NW
_CHUNK = 2048
_NCHUNK = _APW // _CHUNK
_TBL = _NP * _MAXZ


def _build_sc_gather():
    mesh = plsc.VectorSubcoreMesh(
        core_axis_name="c", subcore_axis_name="s",
        num_cores=_NC, num_subcores=_NS)

    @functools.partial(
        pl.kernel,
        out_type=jax.ShapeDtypeStruct((_NP, _N), jnp.float32),
        mesh=mesh,
        compiler_params=pltpu.CompilerParams(
            needs_layout_passes=False, use_tc_tiling_on_sc=True),
        scratch_types=[
            pltpu.VMEM((_TBL,), jnp.float32),
            pltpu.VMEM((_CHUNK,), jnp.int32),
            pltpu.VMEM((_CHUNK,), jnp.int32),
            pltpu.VMEM((_NP, _CHUNK), jnp.float32),
            pltpu.VMEM((_NP, _CHUNK), jnp.float32),
            pltpu.SemaphoreType.DMA,
            pltpu.SemaphoreType.DMA,
            pltpu.SemaphoreType.DMA,
            pltpu.SemaphoreType.DMA,
        ],
    )
    def sc_gather(z_hbm, pt_hbm, out_hbm, pt_v, z_a, z_b, blk_a, blk_b,
                  osem_a, osem_b, zsem_a, zsem_b):
        wid = lax.axis_index("s") * _NC + lax.axis_index("c")
        atom0 = wid * _APW
        pltpu.sync_copy(pt_hbm, pt_v)
        pltpu.async_copy(z_hbm.at[pl.ds(atom0, _CHUNK)], z_a, zsem_a)

        def compute(blk, z_v):
            @plsc.parallel_loop(0, _CHUNK // _L, unroll=1)
            def group_body(a):
                aoff = a * _L
                zvec = z_v[pl.ds(aoff, _L)]
                for j in range(_NP):
                    vals = plsc.load_gather(pt_v, [zvec + (_MAXZ * j)])
                    blk[j, pl.ds(aoff, _L)] = vals

        def pipe_body(g2, carry):
            for blk, osem, z_cur, zsem_cur, z_nxt, zsem_nxt, cidx in (
                    (blk_a, osem_a, z_a, zsem_a, z_b, zsem_b, 2 * g2),
                    (blk_b, osem_b, z_b, zsem_b, z_a, zsem_a, 2 * g2 + 1)):
                @pl.when(cidx + 1 < _NCHUNK)
                def _prefetch():
                    pltpu.async_copy(
                        z_hbm.at[pl.ds(atom0 + (cidx + 1) * _CHUNK, _CHUNK)],
                        z_nxt, zsem_nxt)

                pltpu.make_async_copy(
                    z_hbm.at[pl.ds(atom0, _CHUNK)], z_cur, zsem_cur).wait()

                @pl.when(g2 > 0)
                def _wait():
                    pltpu.make_async_copy(
                        blk, out_hbm.at[:, pl.ds(atom0, _CHUNK)], osem).wait()

                compute(blk, z_cur)
                pltpu.async_copy(
                    blk, out_hbm.at[:, pl.ds(atom0 + cidx * _CHUNK, _CHUNK)],
                    osem)
            return carry

        lax.fori_loop(0, _NCHUNK // 2, pipe_body, 0)
        pltpu.make_async_copy(
            blk_a, out_hbm.at[:, pl.ds(atom0, _CHUNK)], osem_a).wait()
        pltpu.make_async_copy(
            blk_b, out_hbm.at[:, pl.ds(atom0, _CHUNK)], osem_b).wait()

    return sc_gather


_SC_GATHER = _build_sc_gather()


def kernel(Z, p, alpha, chi):
    z32 = Z.astype(jnp.int32)
    pt = p.T.reshape(_TBL)
    outT = _SC_GATHER(z32, pt)
    return (outT.T, alpha, chi)
